# baseline (speedup 1.0000x reference)
import jax
import jax.numpy as jnp
from jax.experimental import pallas as pl


def _copy_body(x_ref, o_ref):
    o_ref[...] = x_ref[...]


def kernel(logits):
    B, T, V = logits.shape
    rows = B * T
    x = logits.reshape(rows, V)
    bt = min(16, rows)
    out = pl.pallas_call(
        _copy_body,
        grid=(pl.cdiv(rows, bt),),
        in_specs=[pl.BlockSpec((bt, V), lambda i: (i, 0))],
        out_specs=pl.BlockSpec((bt, V), lambda i: (i, 0)),
        out_shape=jax.ShapeDtypeStruct((rows, V), logits.dtype),
    )(x)
    return out.reshape(B, T, V)


# speedup vs baseline: 1.1146x; 1.1146x over previous
import jax
import jax.numpy as jnp
from jax.experimental import pallas as pl


def _copy_body(x_ref, o_ref):
    o_ref[...] = x_ref[...]


def kernel(logits):
    B, T, V = logits.shape
    rows = B * T
    x = logits.reshape(rows, V)
    bt = min(96, rows)
    out = pl.pallas_call(
        _copy_body,
        grid=(pl.cdiv(rows, bt),),
        in_specs=[pl.BlockSpec((bt, V), lambda i: (i, 0))],
        out_specs=pl.BlockSpec((bt, V), lambda i: (i, 0)),
        out_shape=jax.ShapeDtypeStruct((rows, V), logits.dtype),
    )(x)
    return out.reshape(B, T, V)
